# baseline (speedup 1.0000x reference)
import jax
import jax.numpy as jnp
from jax import lax
from jax.experimental import pallas as pl
from jax.experimental.pallas import tpu as pltpu

_HW_SPLIT = 2


def _body(x_ref, w_ref, b_ref, out_ref, acc_ref):
    j = pl.program_id(1)
    s = jnp.sum(x_ref[...], axis=1)

    @pl.when(j == 0)
    def _():
        acc_ref[...] = s

    @pl.when(j > 0)
    def _():
        acc_ref[...] = acc_ref[...] + s

    @pl.when(j == _HW_SPLIT - 1)
    def _():
        pooled = acc_ref[...] * (1.0 / (x_ref.shape[1] * _HW_SPLIT))
        logits = lax.dot_general(
            pooled, w_ref[...], (((1,), (1,)), ((), ())),
            preferred_element_type=jnp.float32,
        ) + b_ref[...]
        bb, e = logits.shape
        idx = lax.broadcasted_iota(jnp.int32, (bb, e), 1)
        m1 = jnp.max(logits, axis=1, keepdims=True)
        i1 = jnp.min(jnp.where(logits == m1, idx, e), axis=1, keepdims=True)
        neg = jnp.where(idx == i1, -jnp.inf, logits)
        m2 = jnp.max(neg, axis=1, keepdims=True)
        i2 = jnp.min(jnp.where(neg == m2, idx, e), axis=1, keepdims=True)
        e2 = jnp.exp(m2 - m1)
        denom = 1.0 + e2
        w1 = 1.0 / denom
        w2 = e2 / denom
        out_ref[...] = jnp.where(idx == i1, w1, jnp.where(idx == i2, w2, 0.0))


@jax.jit
def kernel(x, W, b):
    B, C, H, Wd = x.shape
    E = W.shape[0]
    HW = H * Wd
    x3 = jnp.transpose(x, (0, 2, 3, 1)).reshape(B, HW, C)
    b2 = b.reshape(1, E)
    BB = 8
    HWB = HW // _HW_SPLIT
    return pl.pallas_call(
        _body,
        grid=(B // BB, _HW_SPLIT),
        in_specs=[
            pl.BlockSpec((BB, HWB, C), lambda i, j: (i, j, 0)),
            pl.BlockSpec((E, C), lambda i, j: (0, 0)),
            pl.BlockSpec((1, E), lambda i, j: (0, 0)),
        ],
        out_specs=pl.BlockSpec((BB, E), lambda i, j: (i, 0)),
        out_shape=jax.ShapeDtypeStruct((B, E), jnp.float32),
        scratch_shapes=[pltpu.VMEM((BB, C), jnp.float32)],
    )(x3, W, b2)


# speedup vs baseline: 1.0139x; 1.0139x over previous
import jax
import jax.numpy as jnp
from jax import lax
from jax.experimental import pallas as pl


def _body(x_ref, w_ref, b_ref, out_ref):
    s = jnp.sum(x_ref[...], axis=1)
    pooled = s * (1.0 / x_ref.shape[1])
    logits = lax.dot_general(
        pooled, w_ref[...], (((1,), (1,)), ((), ())),
        preferred_element_type=jnp.float32,
    ) + b_ref[...]
    bb, e = logits.shape
    idx = lax.broadcasted_iota(jnp.int32, (bb, e), 1)
    m1 = jnp.max(logits, axis=1, keepdims=True)
    i1 = jnp.min(jnp.where(logits == m1, idx, e), axis=1, keepdims=True)
    neg = jnp.where(idx == i1, -jnp.inf, logits)
    m2 = jnp.max(neg, axis=1, keepdims=True)
    i2 = jnp.min(jnp.where(neg == m2, idx, e), axis=1, keepdims=True)
    e2 = jnp.exp(m2 - m1)
    denom = 1.0 + e2
    w1 = 1.0 / denom
    w2 = e2 / denom
    out_ref[...] = jnp.where(idx == i1, w1, jnp.where(idx == i2, w2, 0.0))


@jax.jit
def kernel(x, W, b):
    B, C, H, Wd = x.shape
    E = W.shape[0]
    HW = H * Wd
    x3 = jnp.transpose(x, (0, 2, 3, 1)).reshape(B, HW, C)
    b2 = b.reshape(1, E)
    BB = 8
    return pl.pallas_call(
        _body,
        grid=(B // BB,),
        in_specs=[
            pl.BlockSpec((BB, HW, C), lambda i: (i, 0, 0)),
            pl.BlockSpec((E, C), lambda i: (0, 0)),
            pl.BlockSpec((1, E), lambda i: (0, 0)),
        ],
        out_specs=pl.BlockSpec((BB, E), lambda i: (i, 0)),
        out_shape=jax.ShapeDtypeStruct((B, E), jnp.float32),
    )(x3, W, b2)
